# baseline (speedup 1.0000x reference)
import dataclasses
import functools

import jax
import jax.numpy as jnp
from jax import lax
from jax.experimental import pallas as pl
from jax.experimental.pallas import tpu as pltpu
from jax.experimental.pallas import tpu_sc as plsc

N = 10000
E = 160000
F = 256
D = 16
G = 64

NC = 2
NS = 16
NW = NC * NS
HF = F // NC
CH = 100
GCH = 32
E_PAD = 161792
ROWS_PER_S = (E // CH) // NS
NPS = N // NS
NPAD = 10240

def _mesh():
    return plsc.VectorSubcoreMesh(core_axis_name="c", subcore_axis_name="s")


def _no_layout_cp():
    cp = pltpu.CompilerParams()
    if "needs_layout_passes" in pltpu.CompilerParams.__dataclass_fields__:
        cp = dataclasses.replace(cp, needs_layout_passes=False)
    return cp


def _bcast_lane(v, j):
    return lax.gather(
        v, jnp.full((16, 1), j, jnp.int32),
        lax.GatherDimensionNumbers(offset_dims=(), collapsed_slice_dims=(0,),
                                   start_index_map=(0,)),
        slice_sizes=(1,), mode=lax.GatherScatterMode.PROMISE_IN_BOUNDS)


def _mm_body(h_ref, wd_ref, ws_ref, pd_ref, ps_ref):
    h = h_ref[...]
    pd_ref[...] = jnp.dot(h, wd_ref[...], preferred_element_type=jnp.float32)
    ps_ref[...] = jnp.dot(h, ws_ref[...], preferred_element_type=jnp.float32)


def _node_matmul(h, Wd, Wsp):
    BM = 1000
    return pl.pallas_call(
        _mm_body,
        grid=(N // BM,),
        in_specs=[pl.BlockSpec((BM, F), lambda i: (i, 0)),
                  pl.BlockSpec((F, 2 * F), lambda i: (0, 0)),
                  pl.BlockSpec((F, 2 * F), lambda i: (0, 0))],
        out_specs=[pl.BlockSpec((BM, 2 * F), lambda i: (i, 0))] * 2,
        out_shape=[jax.ShapeDtypeStruct((N, 2 * F), jnp.float32)] * 2,
    )(h, Wd, Wsp)


def _sc_gather(Pd, Ps, dst_g3, src_g3):
    NCK = E_PAD // GCH
    TRIPS = NCK // NW

    @functools.partial(
        pl.kernel,
        out_type=[jax.ShapeDtypeStruct((NCK, GCH, 2 * F), jnp.float32)] * 2,
        mesh=_mesh(),
        scratch_types=[
            pltpu.VMEM((1, GCH), jnp.int32),
            pltpu.VMEM((1, GCH), jnp.int32),
            pltpu.VMEM((1, GCH), jnp.int32),
            pltpu.VMEM((1, GCH), jnp.int32),
            pltpu.VMEM((GCH, 2 * F), jnp.float32),
            pltpu.VMEM((GCH, 2 * F), jnp.float32),
            pltpu.VMEM((GCH, 2 * F), jnp.float32),
            pltpu.VMEM((GCH, 2 * F), jnp.float32),
            pltpu.SemaphoreType.DMA,
            pltpu.SemaphoreType.DMA,
            pltpu.SemaphoreType.DMA,
            pltpu.SemaphoreType.DMA,
        ],
    )
    def k(pd_hbm, ps_hbm, d_hbm, s_hbm, gd_hbm, gs_hbm,
          di0, di1, si0, si1, bd0, bd1, bs0, bs1,
          semd0, semd1, sems0, sems1):
        wid = lax.axis_index("s") * NC + lax.axis_index("c")
        di = (di0, di1)
        si = (si0, si1)
        bd = (bd0, bd1)
        bs = (bs0, bs1)
        semd = (semd0, semd1)
        sems = (sems0, sems1)

        def issue(tt, b):
            ck = wid + NW * tt
            pltpu.sync_copy(d_hbm.at[ck], di[b])
            pltpu.sync_copy(s_hbm.at[ck], si[b])
            pltpu.async_copy(pd_hbm.at[di[b].at[0]], bd[b], semd[b])
            pltpu.async_copy(ps_hbm.at[si[b].at[0]], bs[b], sems[b])

        def finish(tt, b):
            ck = wid + NW * tt
            pltpu.make_async_copy(pd_hbm.at[di[b].at[0]], bd[b],
                                  semd[b]).wait()
            pltpu.make_async_copy(ps_hbm.at[si[b].at[0]], bs[b],
                                  sems[b]).wait()
            pltpu.sync_copy(bd[b], gd_hbm.at[ck])
            pltpu.sync_copy(bs[b], gs_hbm.at[ck])

        issue(0, 0)
        issue(1, 1)

        @pl.loop(0, TRIPS, step=2)
        def _(t):
            for b in range(2):
                tt = t + b
                finish(tt, b)

                @pl.when(tt + 2 < TRIPS)
                def _():
                    issue(tt + 2, b)

    return k(Pd, Ps, dst_g3, src_g3)


def _edge_body(gd_ref, gs_ref, ea_ref, we_ref, bf_ref, bs_ref, m_ref):
    ez = jnp.dot(ea_ref[...], we_ref[...], preferred_element_type=jnp.float32)
    z = gd_ref[...] + gs_ref[...] + ez
    zf = z[:, :F] + bf_ref[...]
    zs = z[:, F:] + bs_ref[...]
    sig = 1.0 / (1.0 + jnp.exp(-zf))
    sp = jnp.maximum(zs, 0.0) + jnp.log(1.0 + jnp.exp(-jnp.abs(zs)))
    m_ref[...] = sig * sp


def _edge_mlp(Gd, Gs, ea, We, bf_l, bs_l):
    BE = 1000
    return pl.pallas_call(
        _edge_body,
        grid=(E // BE,),
        in_specs=[pl.BlockSpec((BE, 2 * F), lambda i: (i, 0)),
                  pl.BlockSpec((BE, 2 * F), lambda i: (i, 0)),
                  pl.BlockSpec((BE, D), lambda i: (i, 0)),
                  pl.BlockSpec((D, 2 * F), lambda i: (0, 0)),
                  pl.BlockSpec((1, F), lambda i: (0, 0)),
                  pl.BlockSpec((1, F), lambda i: (0, 0))],
        out_specs=pl.BlockSpec((BE, F), lambda i: (i, 0)),
        out_shape=jax.ShapeDtypeStruct((E, F), jnp.float32),
    )(Gd, Gs, ea, We, bf_l, bs_l)


def _sc_scatter_add(m3, dst_s3):
    ZB = 32
    NPP = NPAD // NS

    @functools.partial(
        pl.kernel,
        out_type=jax.ShapeDtypeStruct((NPAD, F), jnp.float32),
        mesh=_mesh(),
        scratch_types=[
            pltpu.VMEM((ROWS_PER_S, CH), jnp.int32),
            pltpu.VMEM((CH, HF), jnp.float32),
            pltpu.VMEM((ZB, HF), jnp.float32),
            pltpu.VMEM_SHARED((NPAD, HF), jnp.float32),
        ],
    )
    def k(m_hbm, d_hbm, out_hbm, idx, buf, zbuf, acc):
        c = lax.axis_index("c")
        s = lax.axis_index("s")

        @pl.loop(0, ZB)
        def _(r):
            for kk in range(HF // 16):
                zbuf.at[r, pl.ds(kk * 16, 16)][...] = jnp.zeros(
                    (16,), jnp.float32)

        @pl.loop(0, NPP, step=ZB)
        def _(r):
            pltpu.sync_copy(zbuf, acc.at[pl.ds(s * NPP + r, ZB)])

        pltpu.sync_copy(d_hbm.at[s], idx)
        plsc.subcore_barrier()

        @pl.loop(0, ROWS_PER_S)
        def _(j):
            pltpu.sync_copy(m_hbm.at[s * ROWS_PER_S + j, :, pl.ds(c * HF, HF)],
                            buf)
            pltpu.sync_copy(buf, acc.at[idx.at[j]], add=True)

        plsc.subcore_barrier()
        pltpu.sync_copy(acc.at[pl.ds(s * NPP, NPP)],
                        out_hbm.at[pl.ds(s * NPP, NPP), pl.ds(c * HF, HF)])

    return k(m3, dst_s3)


def _bn_stats_body(a_ref, o_ref):
    i = pl.program_id(0)

    @pl.when(i == 0)
    def _():
        o_ref[...] = jnp.zeros_like(o_ref)

    a = a_ref[...]
    o_ref[0:1, :] += jnp.sum(a, axis=0, keepdims=True)
    o_ref[1:2, :] += jnp.sum(a * a, axis=0, keepdims=True)


def _bn_apply_body(a_ref, h_ref, st_ref, g_ref, b_ref, o_ref):
    mean = st_ref[0:1, :] * (1.0 / N)
    var = st_ref[1:2, :] * (1.0 / N) - mean * mean
    rstd = lax.rsqrt(var + 1e-5)
    o_ref[...] = (a_ref[...] - mean) * rstd * g_ref[...] + b_ref[...] + h_ref[...]


def _bn_residual(agg, h, g_l, b_l):
    BM = 1000
    stats = pl.pallas_call(
        _bn_stats_body,
        grid=(N // BM,),
        in_specs=[pl.BlockSpec((BM, F), lambda i: (i, 0))],
        out_specs=pl.BlockSpec((2, F), lambda i: (0, 0)),
        out_shape=jax.ShapeDtypeStruct((2, F), jnp.float32),
    )(agg)
    return pl.pallas_call(
        _bn_apply_body,
        grid=(N // BM,),
        in_specs=[pl.BlockSpec((BM, F), lambda i: (i, 0)),
                  pl.BlockSpec((BM, F), lambda i: (i, 0)),
                  pl.BlockSpec((2, F), lambda i: (0, 0)),
                  pl.BlockSpec((1, F), lambda i: (0, 0)),
                  pl.BlockSpec((1, F), lambda i: (0, 0))],
        out_specs=pl.BlockSpec((BM, F), lambda i: (i, 0)),
        out_shape=jax.ShapeDtypeStruct((N, F), jnp.float32),
    )(agg, h, stats, g_l, b_l)


def _sc_segment_max(h, batch):
    NCHUNK = N // 16
    TPS = (NCHUNK + NS - 1) // NS
    GPS = 8
    NRED = G // GPS

    @functools.partial(
        pl.kernel,
        out_type=jax.ShapeDtypeStruct((G, F), jnp.float32),
        mesh=_mesh(),
        compiler_params=_no_layout_cp(),
        scratch_types=[
            pltpu.VMEM((G, HF), jnp.float32),
            pltpu.VMEM((16, HF), jnp.float32),
            pltpu.VMEM((16,), jnp.int32),
            pltpu.VMEM((GPS, HF), jnp.float32),
            pltpu.VMEM((GPS, HF), jnp.float32),
            pltpu.VMEM_SHARED((NS * G, HF), jnp.float32),
        ],
    )
    def k(h_hbm, b_hbm, out_hbm, acc, rowbuf, idbuf, racc, tbuf, shacc):
        c = lax.axis_index("c")
        s = lax.axis_index("s")

        @pl.loop(0, G)
        def _(r):
            for kk in range(HF // 16):
                acc.at[r, pl.ds(kk * 16, 16)][...] = jnp.full(
                    (16,), -jnp.inf, jnp.float32)

        @pl.loop(0, TPS)
        def _(t):
            cid = s + NS * t

            @pl.when(cid < NCHUNK)
            def _():
                pltpu.sync_copy(b_hbm.at[pl.ds(cid * 16, 16)], idbuf)
                pltpu.sync_copy(
                    h_hbm.at[pl.ds(cid * 16, 16), pl.ds(c * HF, HF)], rowbuf)
                ids = idbuf[...]
                for j in range(16):
                    rsp = _bcast_lane(ids, j)
                    for kk in range(HF // 16):
                        colv = kk * 16 + lax.iota(jnp.int32, 16)
                        a = plsc.load_gather(acc, [rsp, colv])
                        r = rowbuf.at[j, pl.ds(kk * 16, 16)][...]
                        plsc.store_scatter(acc, [rsp, colv],
                                           jnp.maximum(a, r))

        pltpu.sync_copy(acc, shacc.at[pl.ds(s * G, G)])
        plsc.subcore_barrier()

        @pl.when(s < NRED)
        def _():
            pltpu.sync_copy(shacc.at[pl.ds(s * GPS, GPS)], racc)

            @pl.loop(1, NS)
            def _(t):
                pltpu.sync_copy(shacc.at[pl.ds(t * G + s * GPS, GPS)], tbuf)
                for rr in range(GPS):
                    for kk in range(HF // 16):
                        sl = (rr, pl.ds(kk * 16, 16))
                        racc.at[*sl][...] = jnp.maximum(racc.at[*sl][...],
                                                        tbuf.at[*sl][...])

            pltpu.sync_copy(
                racc, out_hbm.at[pl.ds(s * GPS, GPS), pl.ds(c * HF, HF)])

    return k(h, batch)


def _head_body(p_ref, w1_ref, b1_ref, g2_ref, b2_ref, w2_ref, bo_ref, o_ref):
    o1 = jnp.dot(p_ref[...], w1_ref[...], preferred_element_type=jnp.float32)
    o1 = jnp.maximum(o1 + b1_ref[...], 0.0)
    mean = jnp.mean(o1, axis=0, keepdims=True)
    var = jnp.mean((o1 - mean) ** 2, axis=0, keepdims=True)
    o1 = (o1 - mean) * lax.rsqrt(var + 1e-5) * g2_ref[...] + b2_ref[...]
    o_ref[...] = jnp.dot(o1, w2_ref[...],
                         preferred_element_type=jnp.float32) + bo_ref[...]


def _head(pooled, W1, b1, g2, b2, W2p, bout):
    DENSE = W1.shape[1]
    return pl.pallas_call(
        _head_body,
        out_shape=jax.ShapeDtypeStruct((G, 128), jnp.float32),
    )(pooled, W1, b1, g2, b2, W2p, bout)


def kernel(x, edge_index, edge_attr, batch, Wf, bf, Ws, bs, gbn, bbn,
           W1, b1, g2, b2, W2, bout):
    src = edge_index[0].astype(jnp.int32)
    dst = edge_index[1].astype(jnp.int32)
    pad = jnp.zeros((E_PAD - E,), jnp.int32)
    dst_g3 = jnp.concatenate([dst, pad]).reshape(E_PAD // GCH, 1, GCH)
    src_g3 = jnp.concatenate([src, pad]).reshape(E_PAD // GCH, 1, GCH)
    dst_s3 = dst.reshape(NS, ROWS_PER_S, CH)

    h = x
    L = Wf.shape[0]
    for l in range(L):
        Wd = jnp.concatenate([Wf[l, 0:F], Ws[l, 0:F]], axis=1)
        Wsp = jnp.concatenate([Wf[l, F:2 * F], Ws[l, F:2 * F]], axis=1)
        We = jnp.concatenate([Wf[l, 2 * F:], Ws[l, 2 * F:]], axis=1)
        Pd, Ps = _node_matmul(h, Wd, Wsp)
        Gd, Gs = _sc_gather(Pd, Ps, dst_g3, src_g3)
        m = _edge_mlp(Gd.reshape(E_PAD, 2 * F), Gs.reshape(E_PAD, 2 * F),
                      edge_attr, We, bf[l].reshape(1, F), bs[l].reshape(1, F))
        agg = _sc_scatter_add(m.reshape(E // CH, CH, F), dst_s3)
        h = _bn_residual(agg, h, gbn[l].reshape(1, F), bbn[l].reshape(1, F))

    pooled = _sc_segment_max(h, batch.astype(jnp.int32))
    out = _head(pooled, W1, b1.reshape(1, -1), g2.reshape(1, -1),
                b2.reshape(1, -1), jnp.pad(W2, ((0, 0), (0, 127))),
                bout.reshape(1, 1))
    return out[:, 0:1]


# speedup vs baseline: 1.0619x; 1.0619x over previous
import dataclasses
import functools

import jax
import jax.numpy as jnp
from jax import lax
from jax.experimental import pallas as pl
from jax.experimental.pallas import tpu as pltpu
from jax.experimental.pallas import tpu_sc as plsc

N = 10000
E = 160000
F = 256
D = 16
G = 64

NC = 2
NS = 16
NW = NC * NS
HF = F // NC
CH = 100
GCH = 32
E_PAD = 161792
ROWS_PER_S = (E // CH) // NS
NPS = N // NS
NPAD = 10240

def _mesh():
    return plsc.VectorSubcoreMesh(core_axis_name="c", subcore_axis_name="s")


def _no_layout_cp():
    cp = pltpu.CompilerParams()
    if "needs_layout_passes" in pltpu.CompilerParams.__dataclass_fields__:
        cp = dataclasses.replace(cp, needs_layout_passes=False)
    return cp


def _bcast_lane(v, j):
    return lax.gather(
        v, jnp.full((16, 1), j, jnp.int32),
        lax.GatherDimensionNumbers(offset_dims=(), collapsed_slice_dims=(0,),
                                   start_index_map=(0,)),
        slice_sizes=(1,), mode=lax.GatherScatterMode.PROMISE_IN_BOUNDS)


def _mm_body(h_ref, wd_ref, ws_ref, pd_ref, ps_ref):
    h = h_ref[...]
    pd_ref[...] = jnp.dot(h, wd_ref[...], preferred_element_type=jnp.float32)
    ps_ref[...] = jnp.dot(h, ws_ref[...], preferred_element_type=jnp.float32)


def _node_matmul(h, Wd, Wsp):
    BM = 1000
    return pl.pallas_call(
        _mm_body,
        grid=(N // BM,),
        in_specs=[pl.BlockSpec((BM, F), lambda i: (i, 0)),
                  pl.BlockSpec((F, 2 * F), lambda i: (0, 0)),
                  pl.BlockSpec((F, 2 * F), lambda i: (0, 0))],
        out_specs=[pl.BlockSpec((BM, 2 * F), lambda i: (i, 0))] * 2,
        out_shape=[jax.ShapeDtypeStruct((N, 2 * F), jnp.float32)] * 2,
    )(h, Wd, Wsp)


def _sc_gather(Pd, Ps, dst_g3, src_g3):
    NCK = E_PAD // GCH
    TRIPS = NCK // NW

    @functools.partial(
        pl.kernel,
        out_type=[jax.ShapeDtypeStruct((NCK, GCH, 2 * F), jnp.float32)] * 2,
        mesh=_mesh(),
        scratch_types=[
            pltpu.VMEM((1, GCH), jnp.int32),
            pltpu.VMEM((1, GCH), jnp.int32),
            pltpu.VMEM((1, GCH), jnp.int32),
            pltpu.VMEM((1, GCH), jnp.int32),
            pltpu.VMEM((GCH, 2 * F), jnp.float32),
            pltpu.VMEM((GCH, 2 * F), jnp.float32),
            pltpu.VMEM((GCH, 2 * F), jnp.float32),
            pltpu.VMEM((GCH, 2 * F), jnp.float32),
            pltpu.SemaphoreType.DMA,
            pltpu.SemaphoreType.DMA,
            pltpu.SemaphoreType.DMA,
            pltpu.SemaphoreType.DMA,
        ],
    )
    def k(pd_hbm, ps_hbm, d_hbm, s_hbm, gd_hbm, gs_hbm,
          di0, di1, si0, si1, bd0, bd1, bs0, bs1,
          semd0, semd1, sems0, sems1):
        wid = lax.axis_index("s") * NC + lax.axis_index("c")
        di = (di0, di1)
        si = (si0, si1)
        bd = (bd0, bd1)
        bs = (bs0, bs1)
        semd = (semd0, semd1)
        sems = (sems0, sems1)

        def issue(tt, b):
            ck = wid + NW * tt
            pltpu.sync_copy(d_hbm.at[ck], di[b])
            pltpu.sync_copy(s_hbm.at[ck], si[b])
            pltpu.async_copy(pd_hbm.at[di[b].at[0]], bd[b], semd[b])
            pltpu.async_copy(ps_hbm.at[si[b].at[0]], bs[b], sems[b])

        def finish(tt, b):
            ck = wid + NW * tt
            pltpu.make_async_copy(pd_hbm.at[di[b].at[0]], bd[b],
                                  semd[b]).wait()
            pltpu.make_async_copy(ps_hbm.at[si[b].at[0]], bs[b],
                                  sems[b]).wait()
            pltpu.sync_copy(bd[b], gd_hbm.at[ck])
            pltpu.sync_copy(bs[b], gs_hbm.at[ck])

        issue(0, 0)
        issue(1, 1)

        @pl.loop(0, TRIPS, step=2)
        def _(t):
            for b in range(2):
                tt = t + b
                finish(tt, b)

                @pl.when(tt + 2 < TRIPS)
                def _():
                    issue(tt + 2, b)

    return k(Pd, Ps, dst_g3, src_g3)


def _edge_body(gd_ref, gs_ref, ea_ref, we_ref, bf_ref, bs_ref, m_ref):
    ez = jnp.dot(ea_ref[...], we_ref[...], preferred_element_type=jnp.float32)
    z = gd_ref[...] + gs_ref[...] + ez
    zf = z[:, :F] + bf_ref[...]
    zs = z[:, F:] + bs_ref[...]
    sig = 1.0 / (1.0 + jnp.exp(-zf))
    sp = jnp.maximum(zs, 0.0) + jnp.log(1.0 + jnp.exp(-jnp.abs(zs)))
    m_ref[...] = sig * sp


def _edge_mlp(Gd, Gs, ea, We, bf_l, bs_l):
    BE = 1000
    return pl.pallas_call(
        _edge_body,
        grid=(E // BE,),
        in_specs=[pl.BlockSpec((BE, 2 * F), lambda i: (i, 0)),
                  pl.BlockSpec((BE, 2 * F), lambda i: (i, 0)),
                  pl.BlockSpec((BE, D), lambda i: (i, 0)),
                  pl.BlockSpec((D, 2 * F), lambda i: (0, 0)),
                  pl.BlockSpec((1, F), lambda i: (0, 0)),
                  pl.BlockSpec((1, F), lambda i: (0, 0))],
        out_specs=pl.BlockSpec((BE, F), lambda i: (i, 0)),
        out_shape=jax.ShapeDtypeStruct((E, F), jnp.float32),
    )(Gd, Gs, ea, We, bf_l, bs_l)


def _sc_scatter_add(m3, dst_s3):
    ZB = 32
    NPP = NPAD // NS

    @functools.partial(
        pl.kernel,
        out_type=jax.ShapeDtypeStruct((NPAD, F), jnp.float32),
        mesh=_mesh(),
        scratch_types=[
            pltpu.VMEM((ROWS_PER_S, CH), jnp.int32),
            pltpu.VMEM((CH, HF), jnp.float32),
            pltpu.VMEM((CH, HF), jnp.float32),
            pltpu.VMEM((ZB, HF), jnp.float32),
            pltpu.VMEM_SHARED((NPAD, HF), jnp.float32),
            pltpu.SemaphoreType.DMA,
            pltpu.SemaphoreType.DMA,
        ],
    )
    def k(m_hbm, d_hbm, out_hbm, idx, buf0, buf1, zbuf, acc, lsem0, lsem1):
        c = lax.axis_index("c")
        s = lax.axis_index("s")
        buf = (buf0, buf1)
        lsem = (lsem0, lsem1)

        @pl.loop(0, ZB)
        def _(r):
            for kk in range(HF // 16):
                zbuf.at[r, pl.ds(kk * 16, 16)][...] = jnp.zeros(
                    (16,), jnp.float32)

        @pl.loop(0, NPP, step=ZB)
        def _(r):
            pltpu.sync_copy(zbuf, acc.at[pl.ds(s * NPP + r, ZB)])

        pltpu.sync_copy(d_hbm.at[s], idx)
        plsc.subcore_barrier()

        def load(j, b):
            pltpu.async_copy(
                m_hbm.at[s * ROWS_PER_S + j, :, pl.ds(c * HF, HF)],
                buf[b], lsem[b])

        def finish(j, b):
            pltpu.make_async_copy(
                m_hbm.at[s * ROWS_PER_S + j, :, pl.ds(c * HF, HF)],
                buf[b], lsem[b]).wait()
            pltpu.sync_copy(buf[b], acc.at[idx.at[j]], add=True)

        load(0, 0)
        load(1, 1)

        @pl.loop(0, ROWS_PER_S, step=2)
        def _(j):
            for b in range(2):
                jj = j + b
                finish(jj, b)

                @pl.when(jj + 2 < ROWS_PER_S)
                def _():
                    load(jj + 2, b)

        plsc.subcore_barrier()
        pltpu.sync_copy(acc.at[pl.ds(s * NPP, NPP)],
                        out_hbm.at[pl.ds(s * NPP, NPP), pl.ds(c * HF, HF)])

    return k(m3, dst_s3)


def _bn_stats_body(a_ref, o_ref):
    i = pl.program_id(0)

    @pl.when(i == 0)
    def _():
        o_ref[...] = jnp.zeros_like(o_ref)

    a = a_ref[...]
    o_ref[0:1, :] += jnp.sum(a, axis=0, keepdims=True)
    o_ref[1:2, :] += jnp.sum(a * a, axis=0, keepdims=True)


def _bn_apply_body(a_ref, h_ref, st_ref, g_ref, b_ref, o_ref):
    mean = st_ref[0:1, :] * (1.0 / N)
    var = st_ref[1:2, :] * (1.0 / N) - mean * mean
    rstd = lax.rsqrt(var + 1e-5)
    o_ref[...] = (a_ref[...] - mean) * rstd * g_ref[...] + b_ref[...] + h_ref[...]


def _bn_residual(agg, h, g_l, b_l):
    BM = 1000
    stats = pl.pallas_call(
        _bn_stats_body,
        grid=(N // BM,),
        in_specs=[pl.BlockSpec((BM, F), lambda i: (i, 0))],
        out_specs=pl.BlockSpec((2, F), lambda i: (0, 0)),
        out_shape=jax.ShapeDtypeStruct((2, F), jnp.float32),
    )(agg)
    return pl.pallas_call(
        _bn_apply_body,
        grid=(N // BM,),
        in_specs=[pl.BlockSpec((BM, F), lambda i: (i, 0)),
                  pl.BlockSpec((BM, F), lambda i: (i, 0)),
                  pl.BlockSpec((2, F), lambda i: (0, 0)),
                  pl.BlockSpec((1, F), lambda i: (0, 0)),
                  pl.BlockSpec((1, F), lambda i: (0, 0))],
        out_specs=pl.BlockSpec((BM, F), lambda i: (i, 0)),
        out_shape=jax.ShapeDtypeStruct((N, F), jnp.float32),
    )(agg, h, stats, g_l, b_l)


def _sc_segment_max(h, batch):
    NCHUNK = N // 16
    TPS = (NCHUNK + NS - 1) // NS
    GPS = 8
    NRED = G // GPS

    @functools.partial(
        pl.kernel,
        out_type=jax.ShapeDtypeStruct((G, F), jnp.float32),
        mesh=_mesh(),
        compiler_params=_no_layout_cp(),
        scratch_types=[
            pltpu.VMEM((G, HF), jnp.float32),
            pltpu.VMEM((16, HF), jnp.float32),
            pltpu.VMEM((16,), jnp.int32),
            pltpu.VMEM((GPS, HF), jnp.float32),
            pltpu.VMEM((GPS, HF), jnp.float32),
            pltpu.VMEM_SHARED((NS * G, HF), jnp.float32),
        ],
    )
    def k(h_hbm, b_hbm, out_hbm, acc, rowbuf, idbuf, racc, tbuf, shacc):
        c = lax.axis_index("c")
        s = lax.axis_index("s")

        @pl.loop(0, G)
        def _(r):
            for kk in range(HF // 16):
                acc.at[r, pl.ds(kk * 16, 16)][...] = jnp.full(
                    (16,), -jnp.inf, jnp.float32)

        @pl.loop(0, TPS)
        def _(t):
            cid = s + NS * t

            @pl.when(cid < NCHUNK)
            def _():
                pltpu.sync_copy(b_hbm.at[pl.ds(cid * 16, 16)], idbuf)
                pltpu.sync_copy(
                    h_hbm.at[pl.ds(cid * 16, 16), pl.ds(c * HF, HF)], rowbuf)
                ids = idbuf[...]
                for j in range(16):
                    rsp = _bcast_lane(ids, j)
                    for kk in range(HF // 16):
                        colv = kk * 16 + lax.iota(jnp.int32, 16)
                        a = plsc.load_gather(acc, [rsp, colv])
                        r = rowbuf.at[j, pl.ds(kk * 16, 16)][...]
                        plsc.store_scatter(acc, [rsp, colv],
                                           jnp.maximum(a, r))

        pltpu.sync_copy(acc, shacc.at[pl.ds(s * G, G)])
        plsc.subcore_barrier()

        @pl.when(s < NRED)
        def _():
            pltpu.sync_copy(shacc.at[pl.ds(s * GPS, GPS)], racc)

            @pl.loop(1, NS)
            def _(t):
                pltpu.sync_copy(shacc.at[pl.ds(t * G + s * GPS, GPS)], tbuf)
                for rr in range(GPS):
                    for kk in range(HF // 16):
                        sl = (rr, pl.ds(kk * 16, 16))
                        racc.at[*sl][...] = jnp.maximum(racc.at[*sl][...],
                                                        tbuf.at[*sl][...])

            pltpu.sync_copy(
                racc, out_hbm.at[pl.ds(s * GPS, GPS), pl.ds(c * HF, HF)])

    return k(h, batch)


def _head_body(p_ref, w1_ref, b1_ref, g2_ref, b2_ref, w2_ref, bo_ref, o_ref):
    o1 = jnp.dot(p_ref[...], w1_ref[...], preferred_element_type=jnp.float32)
    o1 = jnp.maximum(o1 + b1_ref[...], 0.0)
    mean = jnp.mean(o1, axis=0, keepdims=True)
    var = jnp.mean((o1 - mean) ** 2, axis=0, keepdims=True)
    o1 = (o1 - mean) * lax.rsqrt(var + 1e-5) * g2_ref[...] + b2_ref[...]
    o_ref[...] = jnp.dot(o1, w2_ref[...],
                         preferred_element_type=jnp.float32) + bo_ref[...]


def _head(pooled, W1, b1, g2, b2, W2p, bout):
    DENSE = W1.shape[1]
    return pl.pallas_call(
        _head_body,
        out_shape=jax.ShapeDtypeStruct((G, 128), jnp.float32),
    )(pooled, W1, b1, g2, b2, W2p, bout)


def kernel(x, edge_index, edge_attr, batch, Wf, bf, Ws, bs, gbn, bbn,
           W1, b1, g2, b2, W2, bout):
    src = edge_index[0].astype(jnp.int32)
    dst = edge_index[1].astype(jnp.int32)
    pad = jnp.zeros((E_PAD - E,), jnp.int32)
    dst_g3 = jnp.concatenate([dst, pad]).reshape(E_PAD // GCH, 1, GCH)
    src_g3 = jnp.concatenate([src, pad]).reshape(E_PAD // GCH, 1, GCH)
    dst_s3 = dst.reshape(NS, ROWS_PER_S, CH)

    h = x
    L = Wf.shape[0]
    for l in range(L):
        Wd = jnp.concatenate([Wf[l, 0:F], Ws[l, 0:F]], axis=1)
        Wsp = jnp.concatenate([Wf[l, F:2 * F], Ws[l, F:2 * F]], axis=1)
        We = jnp.concatenate([Wf[l, 2 * F:], Ws[l, 2 * F:]], axis=1)
        Pd, Ps = _node_matmul(h, Wd, Wsp)
        Gd, Gs = _sc_gather(Pd, Ps, dst_g3, src_g3)
        m = _edge_mlp(Gd.reshape(E_PAD, 2 * F), Gs.reshape(E_PAD, 2 * F),
                      edge_attr, We, bf[l].reshape(1, F), bs[l].reshape(1, F))
        agg = _sc_scatter_add(m.reshape(E // CH, CH, F), dst_s3)
        h = _bn_residual(agg, h, gbn[l].reshape(1, F), bbn[l].reshape(1, F))

    pooled = _sc_segment_max(h, batch.astype(jnp.int32))
    out = _head(pooled, W1, b1.reshape(1, -1), g2.reshape(1, -1),
                b2.reshape(1, -1), jnp.pad(W2, ((0, 0), (0, 127))),
                bout.reshape(1, 1))
    return out[:, 0:1]
